# baseline (speedup 1.0000x reference)
import functools

import jax
import jax.numpy as jnp
import numpy as np
from jax import lax
from jax.experimental import pallas as pl
from jax.experimental.pallas import tpu as pltpu
from jax.experimental.pallas import tpu_sc as plsc

_ALPHA = 4


@functools.lru_cache(maxsize=None)
def _slow_indices(T: int) -> tuple:
    n = T // _ALPHA
    i = np.arange(n - 1, dtype=np.float32) / np.float32(n - 1)
    lo, hi = np.float32(0.0), np.float32(T - 1)
    vals = np.concatenate([lo * (np.float32(1.0) - i) + hi * i, [hi]])
    return tuple(int(v) for v in vals.astype(np.int32))


def _slow_gather_tc(frames, sel):
    C, T, H, W = frames.shape
    S = len(sel)
    N = C * S

    def body(in_hbm, out_hbm, *scratch):
        bufs = scratch[:N]
        rsem, wsem = scratch[N], scratch[N + 1]
        reads = []
        for i in range(N):
            c, k = i // S, i % S
            cp = pltpu.make_async_copy(
                in_hbm.at[c, sel[k]], bufs[i], rsem.at[i]
            )
            cp.start()
            reads.append(cp)
        writes = []
        for i in range(N):
            c, k = i // S, i % S
            reads[i].wait()
            cp = pltpu.make_async_copy(bufs[i], out_hbm.at[c, k], wsem.at[i])
            cp.start()
            writes.append(cp)
        for cp in writes:
            cp.wait()

    return pl.pallas_call(
        body,
        in_specs=[pl.BlockSpec(memory_space=pl.ANY)],
        out_specs=pl.BlockSpec(memory_space=pl.ANY),
        out_shape=jax.ShapeDtypeStruct((C, S, H, W), frames.dtype),
        scratch_shapes=(
            [pltpu.VMEM((H, W), frames.dtype) for _ in range(N)]
            + [pltpu.SemaphoreType.DMA((N,)), pltpu.SemaphoreType.DMA((N,))]
        ),
    )(frames)


def _fast_copy_sc(frames):
    C, T, H, W = frames.shape
    HH = H // 2

    info = plsc.get_sparse_core_info()
    NW = info.num_cores * info.num_subcores
    n_chunks = C * T * 2
    per_w = n_chunks // NW
    NBUF = 3

    mesh = plsc.VectorSubcoreMesh(core_axis_name="c", subcore_axis_name="s")

    def chunk_coords(chunk):
        f = chunk // 2
        half = chunk % 2
        return f // T, f % T, half * HH

    @functools.partial(
        pl.kernel,
        out_type=jax.ShapeDtypeStruct((C, T, H, W), frames.dtype),
        mesh=mesh,
        scratch_types=(
            [pltpu.VMEM((HH, W), frames.dtype) for _ in range(NBUF)]
            + [pltpu.SemaphoreType.DMA for _ in range(NBUF)]
        ),
    )
    def copy(frames_hbm, fast_hbm, *scratch):
        bufs = scratch[:NBUF]
        sems = scratch[NBUF:]
        wid = lax.axis_index("s") * info.num_cores + lax.axis_index("c")
        coords = [chunk_coords(wid * per_w + j) for j in range(per_w)]
        reads, writes = {}, {}
        for j in range(per_w + NBUF - 1):
            if j < per_w:
                b = j % NBUF
                if j >= NBUF:
                    writes[j - NBUF].wait()
                c_, t_, h0 = coords[j]
                reads[j] = pltpu.async_copy(
                    frames_hbm.at[c_, t_, pl.ds(h0, HH)], bufs[b], sems[b]
                )
            d = j - (NBUF - 1)
            if 0 <= d < per_w:
                b = d % NBUF
                c_, t_, h0 = coords[d]
                reads[d].wait()
                writes[d] = pltpu.async_copy(
                    bufs[b], fast_hbm.at[c_, t_, pl.ds(h0, HH)], sems[b]
                )
        for d in range(max(0, per_w - NBUF), per_w):
            if d in writes and d + NBUF >= per_w:
                writes[d].wait()

    return copy(frames)


def kernel(frames):
    T = frames.shape[1]
    sel = _slow_indices(T)
    fast = _fast_copy_sc(frames)
    slow = _slow_gather_tc(frames, sel)
    return (slow, fast)


# speedup vs baseline: 1.0689x; 1.0689x over previous
import functools

import jax
import jax.numpy as jnp
import numpy as np
from jax import lax
from jax.experimental import pallas as pl
from jax.experimental.pallas import tpu as pltpu
from jax.experimental.pallas import tpu_sc as plsc

_ALPHA = 4


@functools.lru_cache(maxsize=None)
def _slow_indices(T: int) -> tuple:
    n = T // _ALPHA
    i = np.arange(n - 1, dtype=np.float32) / np.float32(n - 1)
    lo, hi = np.float32(0.0), np.float32(T - 1)
    vals = np.concatenate([lo * (np.float32(1.0) - i) + hi * i, [hi]])
    return tuple(int(v) for v in vals.astype(np.int32))


def _copy_body(in_ref, out_ref):
    out_ref[...] = in_ref[...]


def _fast_copy(frames):
    C, T, H, W = frames.shape
    BT = 32
    return pl.pallas_call(
        _copy_body,
        grid=(C, T // BT),
        in_specs=[pl.BlockSpec((1, BT, H, W), lambda c, i: (c, i, 0, 0))],
        out_specs=pl.BlockSpec((1, BT, H, W), lambda c, i: (c, i, 0, 0)),
        out_shape=jax.ShapeDtypeStruct((C, T, H, W), frames.dtype),
        compiler_params=pltpu.CompilerParams(
            dimension_semantics=("arbitrary", "arbitrary")
        ),
    )(frames)


def _slow_gather_sc(frames, sel):
    C, T, H, W = frames.shape
    S = len(sel)
    HH = H // 2

    info = plsc.get_sparse_core_info()
    NW = info.num_cores * info.num_subcores
    n_chunks = C * S * 2
    per_w = n_chunks // NW

    mesh = plsc.VectorSubcoreMesh(core_axis_name="c", subcore_axis_name="s")

    def chunk_coords(chunk):
        r = chunk // 2
        half = chunk % 2
        ch = r // S
        k = r % S
        src_t = functools.reduce(
            lambda acc, i: jnp.where(k == i, sel[i], acc),
            range(S),
            jnp.int32(0),
        )
        return ch, k, src_t, half * HH

    @functools.partial(
        pl.kernel,
        out_type=jax.ShapeDtypeStruct((C, S, H, W), frames.dtype),
        mesh=mesh,
        scratch_types=[
            pltpu.VMEM((HH, W), frames.dtype),
            pltpu.VMEM((HH, W), frames.dtype),
            pltpu.VMEM((HH, W), frames.dtype),
            pltpu.SemaphoreType.DMA,
            pltpu.SemaphoreType.DMA,
            pltpu.SemaphoreType.DMA,
        ],
    )
    def gather(frames_hbm, slow_hbm, buf0, buf1, buf2, sem0, sem1, sem2):
        wid = lax.axis_index("s") * info.num_cores + lax.axis_index("c")
        bufs = (buf0, buf1, buf2)
        sems = (sem0, sem1, sem2)
        coords = [chunk_coords(wid * per_w + j) for j in range(per_w)]
        reads = [
            pltpu.async_copy(
                frames_hbm.at[c_, t_, pl.ds(h0, HH)], bufs[j], sems[j]
            )
            for j, (c_, _, t_, h0) in enumerate(coords)
        ]
        writes = []
        for j, (c_, k_, _, h0) in enumerate(coords):
            reads[j].wait()
            writes.append(
                pltpu.async_copy(
                    bufs[j], slow_hbm.at[c_, k_, pl.ds(h0, HH)], sems[j]
                )
            )
        for wr in writes:
            wr.wait()

    return gather(frames)


def kernel(frames):
    T = frames.shape[1]
    sel = _slow_indices(T)
    slow = _slow_gather_sc(frames, sel)
    fast = _fast_copy(frames)
    return (slow, fast)
